# Initial kernel scaffold; baseline (speedup 1.0000x reference)
import jax
import jax.numpy as jnp
from jax.experimental import pallas as pl


def kernel(x, edge_index, edge_attr, W_l, b_l, W_r, b_r, W_e, att, bias_out, gamma, beta):
    raise NotImplementedError("write your pallas kernel here")


# speedup vs baseline: 12.4704x; 12.4704x over previous
import functools

import jax
import jax.numpy as jnp
from jax import lax
from jax.experimental import pallas as pl
from jax.experimental.pallas import tpu as pltpu
from jax.experimental.pallas import tpu_sc as plsc

_N = 10000
_E = 320000
_D_IN = 128
_H = 4
_C = 32
_HC = _H * _C
_D_E = 16
_NEG = 0.2

_info = plsc.get_sparse_core_info()
_NC, _NS, _L = _info.num_cores, _info.num_subcores, _info.num_lanes
_NW = _NC * _NS
_CH = 128
_GRAN = _NW * _CH

_N_PAD = 10240
_E2 = _E + _N
_E2P = ((_E2 + _GRAN - 1) // _GRAN) * _GRAN
_EP = ((_E + _GRAN - 1) // _GRAN) * _GRAN


def _make_sc_gather(V, D, B):
    bpw = B // _NW
    nch = bpw // _CH
    mesh = plsc.VectorSubcoreMesh(core_axis_name="c", subcore_axis_name="s")

    @functools.partial(
        pl.kernel, mesh=mesh,
        out_type=jax.ShapeDtypeStruct((B, D), jnp.float32),
        scratch_types=[
            pltpu.VMEM((_CH,), jnp.int32),
            pltpu.VMEM((_CH, D), jnp.float32),
            pltpu.SemaphoreType.DMA,
        ],
    )
    def k(table_hbm, idx_hbm, out_hbm, idx_v, rows_v, sem):
        wid = lax.axis_index("s") * _NC + lax.axis_index("c")
        base = wid * bpw

        def body(j, carry):
            off = base + j * _CH
            pltpu.sync_copy(idx_hbm.at[pl.ds(off, _CH)], idx_v)
            pltpu.async_copy(table_hbm.at[idx_v], rows_v, sem).wait()
            pltpu.sync_copy(rows_v, out_hbm.at[pl.ds(off, _CH)])
            return carry

        lax.fori_loop(0, nch, body, 0)

    return k


def _make_sc_scatter_add(D, B):
    bpw = B // _NW
    nch = bpw // _CH
    rps = _N_PAD // _NS
    mesh = plsc.VectorSubcoreMesh(core_axis_name="c", subcore_axis_name="s")

    @functools.partial(
        pl.kernel, mesh=mesh,
        out_type=jax.ShapeDtypeStruct((_NC * _N_PAD, D), jnp.float32),
        scratch_types=[
            pltpu.VMEM((_CH,), jnp.int32),
            pltpu.VMEM((_CH, D), jnp.float32),
            pltpu.VMEM_SHARED((_N_PAD, D), jnp.float32),
        ],
    )
    def k(vals_hbm, idx_hbm, zeros_hbm, out_hbm, idx_v, vals_v, shared):
        cid = lax.axis_index("c")
        sid = lax.axis_index("s")
        wid = sid * _NC + cid
        base = wid * bpw

        pltpu.sync_copy(zeros_hbm.at[pl.ds(sid * rps, rps)],
                        shared.at[pl.ds(sid * rps, rps)])
        plsc.subcore_barrier()

        def step(j, carry):
            off = base + j * _CH
            pltpu.sync_copy(idx_hbm.at[pl.ds(off, _CH)], idx_v)
            pltpu.sync_copy(vals_hbm.at[pl.ds(off, _CH)], vals_v)
            pltpu.sync_copy(vals_v, shared.at[idx_v], add=True)
            return carry

        lax.fori_loop(0, nch, step, 0)
        plsc.subcore_barrier()
        pltpu.sync_copy(shared.at[pl.ds(sid * rps, rps)],
                        out_hbm.at[pl.ds(cid * _N_PAD + sid * rps, rps)])

    return k


_BN = 1024
_BE = 512


def _tc_transform(x_pad, W_l, b_l, W_r, b_r, stats_p):
    grid = _N_PAD // _BN

    def body(x_ref, wl_ref, bl_ref, wr_ref, br_ref, s0_ref, s1_ref,
             xl_ref, xr_ref, la_ref):
        xb = x_ref[...]
        xl_ref[...] = jnp.dot(xb, wl_ref[...],
                              preferred_element_type=jnp.float32) + bl_ref[...]
        xr_ref[...] = jnp.dot(xb, wr_ref[...],
                              preferred_element_type=jnp.float32) + br_ref[...]
        st = s0_ref[...] + s1_ref[...]
        attr_sum = st[:, :_D_E]
        deg = st[:, _D_E:_D_E + 1]
        la_ref[...] = attr_sum / jnp.maximum(deg, 1.0)

    return pl.pallas_call(
        body,
        grid=(grid,),
        in_specs=[
            pl.BlockSpec((_BN, _D_IN), lambda i: (i, 0)),
            pl.BlockSpec((_D_IN, _HC), lambda i: (0, 0)),
            pl.BlockSpec((1, _HC), lambda i: (0, 0)),
            pl.BlockSpec((_D_IN, _HC), lambda i: (0, 0)),
            pl.BlockSpec((1, _HC), lambda i: (0, 0)),
            pl.BlockSpec((_BN, _HC), lambda i: (i, 0)),
            pl.BlockSpec((_BN, _HC), lambda i: (i + _N_PAD // _BN, 0)),
        ],
        out_specs=[
            pl.BlockSpec((_BN, _HC), lambda i: (i, 0)),
            pl.BlockSpec((_BN, _HC), lambda i: (i, 0)),
            pl.BlockSpec((_BN, _D_E), lambda i: (i, 0)),
        ],
        out_shape=[
            jax.ShapeDtypeStruct((_N_PAD, _HC), jnp.float32),
            jax.ShapeDtypeStruct((_N_PAD, _HC), jnp.float32),
            jax.ShapeDtypeStruct((_N_PAD, _D_E), jnp.float32),
        ],
    )(x_pad, W_l, b_l.reshape(1, _HC), W_r, b_r.reshape(1, _HC), stats_p, stats_p)


def _tc_logits(gl, gr, ea2p, W_e, att_row, sel):
    grid = _E2P // _BE

    def body(gl_ref, gr_ref, ea_ref, we_ref, att_ref, sel_ref,
             lg_ref, bm_ref):
        i = pl.program_id(0)
        m = gl_ref[...] + gr_ref[...] + jnp.dot(
            ea_ref[...], we_ref[...], preferred_element_type=jnp.float32)
        m = jnp.where(m >= 0, m, _NEG * m)
        md = m * att_ref[...]
        lg = jnp.dot(md, sel_ref[...], preferred_element_type=jnp.float32)
        rows = i * _BE + lax.broadcasted_iota(jnp.int32, (_BE, 16), 0)
        lg = jnp.where(rows < _E2, lg, -1e30)
        lg_ref[...] = lg
        bm_ref[...] = jnp.max(lg, axis=0, keepdims=True)[None]

    return pl.pallas_call(
        body,
        grid=(grid,),
        in_specs=[
            pl.BlockSpec((_BE, _HC), lambda i: (i, 0)),
            pl.BlockSpec((_BE, _HC), lambda i: (i, 0)),
            pl.BlockSpec((_BE, _D_E), lambda i: (i, 0)),
            pl.BlockSpec((_D_E, _HC), lambda i: (0, 0)),
            pl.BlockSpec((1, _HC), lambda i: (0, 0)),
            pl.BlockSpec((_HC, 16), lambda i: (0, 0)),
        ],
        out_specs=[
            pl.BlockSpec((_BE, 16), lambda i: (i, 0)),
            pl.BlockSpec((1, 1, 16), lambda i: (i, 0, 0)),
        ],
        out_shape=[
            jax.ShapeDtypeStruct((_E2P, 16), jnp.float32),
            jax.ShapeDtypeStruct((grid, 1, 16), jnp.float32),
        ],
    )(gl, gr, ea2p, W_e, att_row, sel)


def _tc_exp(logits, blockmax):
    grid = _E2P // _BE
    nb = grid

    def body(lg_ref, bm_ref, ex_ref):
        lmax = jnp.max(bm_ref[...], axis=(0, 1))[None, :]
        ex16 = jnp.exp(lg_ref[...] - lmax)
        ex_ref[...] = jnp.concatenate(
            [ex16, jnp.zeros((_BE, _HC - 16), jnp.float32)], axis=1)

    return pl.pallas_call(
        body,
        grid=(grid,),
        in_specs=[
            pl.BlockSpec((_BE, 16), lambda i: (i, 0)),
            pl.BlockSpec((nb, 1, 16), lambda i: (0, 0, 0)),
        ],
        out_specs=pl.BlockSpec((_BE, _HC), lambda i: (i, 0)),
        out_shape=jax.ShapeDtypeStruct((_E2P, _HC), jnp.float32),
    )(logits, blockmax)


def _tc_alpha_vals(ex, denomg, gl, expand):
    grid = _E2P // _BE

    def body(ex_ref, dg_ref, gl_ref, exp_ref, al_ref, v_ref):
        alpha = ex_ref[...][:, :16] / (dg_ref[...][:, :16] + 1e-16)
        al_ref[...] = alpha
        aw = jnp.dot(alpha, exp_ref[...], preferred_element_type=jnp.float32)
        v_ref[...] = gl_ref[...] * aw

    return pl.pallas_call(
        body,
        grid=(grid,),
        in_specs=[
            pl.BlockSpec((_BE, _HC), lambda i: (i, 0)),
            pl.BlockSpec((_BE, _HC), lambda i: (i, 0)),
            pl.BlockSpec((_BE, _HC), lambda i: (i, 0)),
            pl.BlockSpec((16, _HC), lambda i: (0, 0)),
        ],
        out_specs=[
            pl.BlockSpec((_BE, 16), lambda i: (i, 0)),
            pl.BlockSpec((_BE, _HC), lambda i: (i, 0)),
        ],
        out_shape=[
            jax.ShapeDtypeStruct((_E2P, 16), jnp.float32),
            jax.ShapeDtypeStruct((_E2P, _HC), jnp.float32),
        ],
    )(ex, denomg, gl, expand)


def _tc_final(out_p, bias_row, gamma_row, beta_row):
    grid = _N_PAD // _BN

    def body(o0_ref, o1_ref, b_ref, g_ref, be_ref, hn_ref):
        h = o0_ref[...] + o1_ref[...] + b_ref[...]
        h = h * jax.nn.sigmoid(h)
        mu = jnp.mean(h, axis=1, keepdims=True)
        var = jnp.mean((h - mu) ** 2, axis=1, keepdims=True)
        hn_ref[...] = (h - mu) / jnp.sqrt(var + 1e-5) * g_ref[...] + be_ref[...]

    return pl.pallas_call(
        body,
        grid=(grid,),
        in_specs=[
            pl.BlockSpec((_BN, _HC), lambda i: (i, 0)),
            pl.BlockSpec((_BN, _HC), lambda i: (i + _N_PAD // _BN, 0)),
            pl.BlockSpec((1, _HC), lambda i: (0, 0)),
            pl.BlockSpec((1, _HC), lambda i: (0, 0)),
            pl.BlockSpec((1, _HC), lambda i: (0, 0)),
        ],
        out_specs=pl.BlockSpec((_BN, _HC), lambda i: (i, 0)),
        out_shape=jax.ShapeDtypeStruct((_N_PAD, _HC), jnp.float32),
    )(out_p, out_p, bias_row, gamma_row, beta_row)


def kernel(x, edge_index, edge_attr, W_l, b_l, W_r, b_r, W_e, att, bias_out,
           gamma, beta):
    src, dst = edge_index[0], edge_index[1]

    s_vals = jnp.concatenate(
        [edge_attr, jnp.ones((_E, _D_E), jnp.float32),
         jnp.zeros((_E, _HC - 2 * _D_E), jnp.float32)], axis=1)
    s_vals = jnp.pad(s_vals, ((0, _EP - _E), (0, 0)))
    dst_p = jnp.pad(dst, (0, _EP - _E))
    zeros128 = jnp.zeros((_N_PAD, _HC), jnp.float32)
    stats_p = _make_sc_scatter_add(_HC, _EP)(s_vals, dst_p, zeros128)

    x_pad = jnp.pad(x, ((0, _N_PAD - _N), (0, 0)))
    x_l, x_r, loop_attr = _tc_transform(x_pad, W_l, b_l, W_r, b_r, stats_p)

    loop_idx = jnp.arange(_N, dtype=src.dtype)
    src2 = jnp.pad(jnp.concatenate([src, loop_idx]), (0, _E2P - _E2))
    dst2 = jnp.pad(jnp.concatenate([dst, loop_idx]), (0, _E2P - _E2))
    ea2 = jnp.pad(jnp.concatenate([edge_attr, loop_attr[:_N]], axis=0),
                  ((0, _E2P - _E2), (0, 0)))

    gather_nodes = _make_sc_gather(_N_PAD, _HC, _E2P)
    gl = gather_nodes(x_l, src2)
    gr = gather_nodes(x_r, dst2)

    att_row = att.reshape(1, _HC)
    hsel = (jnp.arange(_HC, dtype=jnp.int32)[:, None] // _C
            == jnp.arange(16, dtype=jnp.int32)[None, :]).astype(jnp.float32)
    logits, blockmax = _tc_logits(gl, gr, ea2, W_e, att_row, hsel)
    ex = _tc_exp(logits, blockmax)

    denom_p = _make_sc_scatter_add(_HC, _E2P)(ex, dst2, zeros128)
    denom = denom_p[:_N_PAD] + denom_p[_N_PAD:]

    denomg = _make_sc_gather(_N_PAD, _HC, _E2P)(denom, dst2)

    expand = (jnp.arange(16, dtype=jnp.int32)[:, None]
              == jnp.arange(_HC, dtype=jnp.int32)[None, :] // _C
              ).astype(jnp.float32)
    alpha_full, vals = _tc_alpha_vals(ex, denomg, gl, expand)

    out_p = _make_sc_scatter_add(_HC, _E2P)(vals, dst2, zeros128)

    hn = _tc_final(out_p, bias_out.reshape(1, _HC), gamma.reshape(1, _HC),
                   beta.reshape(1, _HC))

    return hn[:_N], alpha_full[:_E2, :_H]


# speedup vs baseline: 12.8404x; 1.0297x over previous
import functools

import jax
import jax.numpy as jnp
from jax import lax
from jax.experimental import pallas as pl
from jax.experimental.pallas import tpu as pltpu
from jax.experimental.pallas import tpu_sc as plsc

_N = 10000
_E = 320000
_D_IN = 128
_H = 4
_C = 32
_HC = _H * _C
_D_E = 16
_NEG = 0.2

_info = plsc.get_sparse_core_info()
_NC, _NS, _L = _info.num_cores, _info.num_subcores, _info.num_lanes
_NW = _NC * _NS
_CH = 128
_GRAN = _NW * _CH

_N_PAD = 10240
_E2 = _E + _N
_E2P = ((_E2 + _GRAN - 1) // _GRAN) * _GRAN
_EP = ((_E + _GRAN - 1) // _GRAN) * _GRAN


def _make_sc_gather(V, D, B):
    bpw = B // _NW
    nch = bpw // _CH
    mesh = plsc.VectorSubcoreMesh(core_axis_name="c", subcore_axis_name="s")

    @functools.partial(
        pl.kernel, mesh=mesh,
        out_type=jax.ShapeDtypeStruct((B, D), jnp.float32),
        scratch_types=[
            pltpu.VMEM((bpw,), jnp.int32),
            pltpu.VMEM((_CH, D), jnp.float32),
            pltpu.SemaphoreType.DMA,
        ],
    )
    def k(table_hbm, idx_hbm, out_hbm, idx_v, rows_v, sem):
        wid = lax.axis_index("s") * _NC + lax.axis_index("c")
        base = wid * bpw
        pltpu.sync_copy(idx_hbm.at[pl.ds(base, bpw)], idx_v)

        def body(j, carry):
            off = j * _CH
            pltpu.async_copy(table_hbm.at[idx_v.at[pl.ds(off, _CH)]],
                             rows_v, sem).wait()
            pltpu.sync_copy(rows_v, out_hbm.at[pl.ds(base + off, _CH)])
            return carry

        lax.fori_loop(0, nch, body, 0)

    return k


def _make_sc_scatter_add(D, B):
    bpw = B // _NW
    nch = bpw // _CH
    rps = _N_PAD // _NS
    mesh = plsc.VectorSubcoreMesh(core_axis_name="c", subcore_axis_name="s")

    @functools.partial(
        pl.kernel, mesh=mesh,
        out_type=jax.ShapeDtypeStruct((_NC * _N_PAD, D), jnp.float32),
        scratch_types=[
            pltpu.VMEM((_CH,), jnp.int32),
            pltpu.VMEM((_CH, D), jnp.float32),
            pltpu.VMEM_SHARED((_N_PAD, D), jnp.float32),
        ],
    )
    def k(vals_hbm, idx_hbm, zeros_hbm, out_hbm, idx_v, vals_v, shared):
        cid = lax.axis_index("c")
        sid = lax.axis_index("s")
        wid = sid * _NC + cid
        base = wid * bpw

        pltpu.sync_copy(zeros_hbm.at[pl.ds(sid * rps, rps)],
                        shared.at[pl.ds(sid * rps, rps)])
        plsc.subcore_barrier()

        def step(j, carry):
            off = base + j * _CH
            pltpu.sync_copy(idx_hbm.at[pl.ds(off, _CH)], idx_v)
            pltpu.sync_copy(vals_hbm.at[pl.ds(off, _CH)], vals_v)
            pltpu.sync_copy(vals_v, shared.at[idx_v], add=True)
            return carry

        lax.fori_loop(0, nch, step, 0)
        plsc.subcore_barrier()
        pltpu.sync_copy(shared.at[pl.ds(sid * rps, rps)],
                        out_hbm.at[pl.ds(cid * _N_PAD + sid * rps, rps)])

    return k


_BN = 1024
_BE = 512


def _tc_transform(x_pad, W_l, b_l, W_r, b_r, stats_p):
    grid = _N_PAD // _BN

    def body(x_ref, wl_ref, bl_ref, wr_ref, br_ref, s0_ref, s1_ref,
             xl_ref, xr_ref, la_ref):
        xb = x_ref[...]
        xl_ref[...] = jnp.dot(xb, wl_ref[...],
                              preferred_element_type=jnp.float32) + bl_ref[...]
        xr_ref[...] = jnp.dot(xb, wr_ref[...],
                              preferred_element_type=jnp.float32) + br_ref[...]
        st = s0_ref[...] + s1_ref[...]
        attr_sum = st[:, :_D_E]
        deg = st[:, _D_E:_D_E + 1]
        la_ref[...] = attr_sum / jnp.maximum(deg, 1.0)

    return pl.pallas_call(
        body,
        grid=(grid,),
        in_specs=[
            pl.BlockSpec((_BN, _D_IN), lambda i: (i, 0)),
            pl.BlockSpec((_D_IN, _HC), lambda i: (0, 0)),
            pl.BlockSpec((1, _HC), lambda i: (0, 0)),
            pl.BlockSpec((_D_IN, _HC), lambda i: (0, 0)),
            pl.BlockSpec((1, _HC), lambda i: (0, 0)),
            pl.BlockSpec((_BN, _HC), lambda i: (i, 0)),
            pl.BlockSpec((_BN, _HC), lambda i: (i + _N_PAD // _BN, 0)),
        ],
        out_specs=[
            pl.BlockSpec((_BN, _HC), lambda i: (i, 0)),
            pl.BlockSpec((_BN, _HC), lambda i: (i, 0)),
            pl.BlockSpec((_BN, _D_E), lambda i: (i, 0)),
        ],
        out_shape=[
            jax.ShapeDtypeStruct((_N_PAD, _HC), jnp.float32),
            jax.ShapeDtypeStruct((_N_PAD, _HC), jnp.float32),
            jax.ShapeDtypeStruct((_N_PAD, _D_E), jnp.float32),
        ],
    )(x_pad, W_l, b_l.reshape(1, _HC), W_r, b_r.reshape(1, _HC), stats_p, stats_p)


def _tc_logits(gl, gr, ea2p, W_e, att_row, sel):
    grid = _E2P // _BE

    def body(gl_ref, gr_ref, ea_ref, we_ref, att_ref, sel_ref,
             lg_ref, bm_ref):
        i = pl.program_id(0)
        m = gl_ref[...] + gr_ref[...] + jnp.dot(
            ea_ref[...], we_ref[...], preferred_element_type=jnp.float32)
        m = jnp.where(m >= 0, m, _NEG * m)
        md = m * att_ref[...]
        lg = jnp.dot(md, sel_ref[...], preferred_element_type=jnp.float32)
        rows = i * _BE + lax.broadcasted_iota(jnp.int32, (_BE, 16), 0)
        lg = jnp.where(rows < _E2, lg, -1e30)
        lg_ref[...] = lg
        bm_ref[...] = jnp.max(lg, axis=0, keepdims=True)[None]

    return pl.pallas_call(
        body,
        grid=(grid,),
        in_specs=[
            pl.BlockSpec((_BE, _HC), lambda i: (i, 0)),
            pl.BlockSpec((_BE, _HC), lambda i: (i, 0)),
            pl.BlockSpec((_BE, _D_E), lambda i: (i, 0)),
            pl.BlockSpec((_D_E, _HC), lambda i: (0, 0)),
            pl.BlockSpec((1, _HC), lambda i: (0, 0)),
            pl.BlockSpec((_HC, 16), lambda i: (0, 0)),
        ],
        out_specs=[
            pl.BlockSpec((_BE, 16), lambda i: (i, 0)),
            pl.BlockSpec((1, 1, 16), lambda i: (i, 0, 0)),
        ],
        out_shape=[
            jax.ShapeDtypeStruct((_E2P, 16), jnp.float32),
            jax.ShapeDtypeStruct((grid, 1, 16), jnp.float32),
        ],
    )(gl, gr, ea2p, W_e, att_row, sel)


def _tc_exp(logits, blockmax):
    grid = _E2P // _BE
    nb = grid

    def body(lg_ref, bm_ref, ex_ref):
        lmax = jnp.max(bm_ref[...], axis=(0, 1))[None, :]
        ex16 = jnp.exp(lg_ref[...] - lmax)
        ex_ref[...] = jnp.concatenate(
            [ex16, jnp.zeros((_BE, _HC - 16), jnp.float32)], axis=1)

    return pl.pallas_call(
        body,
        grid=(grid,),
        in_specs=[
            pl.BlockSpec((_BE, 16), lambda i: (i, 0)),
            pl.BlockSpec((nb, 1, 16), lambda i: (0, 0, 0)),
        ],
        out_specs=pl.BlockSpec((_BE, _HC), lambda i: (i, 0)),
        out_shape=jax.ShapeDtypeStruct((_E2P, _HC), jnp.float32),
    )(logits, blockmax)


def _tc_alpha_vals(ex, denomg, gl, expand):
    grid = _E2P // _BE

    def body(ex_ref, dg_ref, gl_ref, exp_ref, al_ref, v_ref):
        alpha = ex_ref[...][:, :16] / (dg_ref[...][:, :16] + 1e-16)
        al_ref[...] = alpha
        aw = jnp.dot(alpha, exp_ref[...], preferred_element_type=jnp.float32)
        v_ref[...] = gl_ref[...] * aw

    return pl.pallas_call(
        body,
        grid=(grid,),
        in_specs=[
            pl.BlockSpec((_BE, _HC), lambda i: (i, 0)),
            pl.BlockSpec((_BE, _HC), lambda i: (i, 0)),
            pl.BlockSpec((_BE, _HC), lambda i: (i, 0)),
            pl.BlockSpec((16, _HC), lambda i: (0, 0)),
        ],
        out_specs=[
            pl.BlockSpec((_BE, 16), lambda i: (i, 0)),
            pl.BlockSpec((_BE, _HC), lambda i: (i, 0)),
        ],
        out_shape=[
            jax.ShapeDtypeStruct((_E2P, 16), jnp.float32),
            jax.ShapeDtypeStruct((_E2P, _HC), jnp.float32),
        ],
    )(ex, denomg, gl, expand)


def _tc_final(out_p, bias_row, gamma_row, beta_row):
    grid = _N_PAD // _BN

    def body(o0_ref, o1_ref, b_ref, g_ref, be_ref, hn_ref):
        h = o0_ref[...] + o1_ref[...] + b_ref[...]
        h = h * jax.nn.sigmoid(h)
        mu = jnp.mean(h, axis=1, keepdims=True)
        var = jnp.mean((h - mu) ** 2, axis=1, keepdims=True)
        hn_ref[...] = (h - mu) / jnp.sqrt(var + 1e-5) * g_ref[...] + be_ref[...]

    return pl.pallas_call(
        body,
        grid=(grid,),
        in_specs=[
            pl.BlockSpec((_BN, _HC), lambda i: (i, 0)),
            pl.BlockSpec((_BN, _HC), lambda i: (i + _N_PAD // _BN, 0)),
            pl.BlockSpec((1, _HC), lambda i: (0, 0)),
            pl.BlockSpec((1, _HC), lambda i: (0, 0)),
            pl.BlockSpec((1, _HC), lambda i: (0, 0)),
        ],
        out_specs=pl.BlockSpec((_BN, _HC), lambda i: (i, 0)),
        out_shape=jax.ShapeDtypeStruct((_N_PAD, _HC), jnp.float32),
    )(out_p, out_p, bias_row, gamma_row, beta_row)


def kernel(x, edge_index, edge_attr, W_l, b_l, W_r, b_r, W_e, att, bias_out,
           gamma, beta):
    src, dst = edge_index[0], edge_index[1]

    s_vals = jnp.concatenate(
        [edge_attr, jnp.ones((_E, _D_E), jnp.float32),
         jnp.zeros((_E, _HC - 2 * _D_E), jnp.float32)], axis=1)
    s_vals = jnp.pad(s_vals, ((0, _EP - _E), (0, 0)))
    dst_p = jnp.pad(dst, (0, _EP - _E))
    zeros128 = jnp.zeros((_N_PAD, _HC), jnp.float32)
    stats_p = _make_sc_scatter_add(_HC, _EP)(s_vals, dst_p, zeros128)

    x_pad = jnp.pad(x, ((0, _N_PAD - _N), (0, 0)))
    x_l, x_r, loop_attr = _tc_transform(x_pad, W_l, b_l, W_r, b_r, stats_p)

    loop_idx = jnp.arange(_N, dtype=src.dtype)
    src2 = jnp.pad(jnp.concatenate([src, loop_idx]), (0, _E2P - _E2))
    dst2 = jnp.pad(jnp.concatenate([dst, loop_idx]), (0, _E2P - _E2))
    ea2 = jnp.pad(jnp.concatenate([edge_attr, loop_attr[:_N]], axis=0),
                  ((0, _E2P - _E2), (0, 0)))

    gather_nodes = _make_sc_gather(_N_PAD, _HC, _E2P)
    gl = gather_nodes(x_l, src2)
    gr = gather_nodes(x_r, dst2)

    att_row = att.reshape(1, _HC)
    hsel = (jnp.arange(_HC, dtype=jnp.int32)[:, None] // _C
            == jnp.arange(16, dtype=jnp.int32)[None, :]).astype(jnp.float32)
    logits, blockmax = _tc_logits(gl, gr, ea2, W_e, att_row, hsel)
    ex = _tc_exp(logits, blockmax)

    denom_p = _make_sc_scatter_add(_HC, _E2P)(ex, dst2, zeros128)
    denom = denom_p[:_N_PAD] + denom_p[_N_PAD:]

    denomg = _make_sc_gather(_N_PAD, _HC, _E2P)(denom, dst2)

    expand = (jnp.arange(16, dtype=jnp.int32)[:, None]
              == jnp.arange(_HC, dtype=jnp.int32)[None, :] // _C
              ).astype(jnp.float32)
    alpha_full, vals = _tc_alpha_vals(ex, denomg, gl, expand)

    out_p = _make_sc_scatter_add(_HC, _E2P)(vals, dst2, zeros128)

    hn = _tc_final(out_p, bias_out.reshape(1, _HC), gamma.reshape(1, _HC),
                   beta.reshape(1, _HC))

    return hn[:_N], alpha_full[:_E2, :_H]
